# baseline (speedup 1.0000x reference)
import functools

import jax
import jax.numpy as jnp
from jax import lax
from jax.experimental import pallas as pl
from jax.experimental.pallas import tpu as pltpu
from jax.experimental.pallas import tpu_sc as plsc

N = 10000
E = 320000
D_IN = 128
HID = 16
D_OUT = 128

NC = 2
NS = 16
NW = NC * NS
BLK = 512
NBLK = 20
EPT = NBLK * BLK
PAD_E = NW * EPT
ROWS_PT = 632
ACC_N = ROWS_PT * NS

_MESH = plsc.VectorSubcoreMesh(core_axis_name="c", subcore_axis_name="s")
_PARAMS = pltpu.CompilerParams(use_tc_tiling_on_sc=False, needs_layout_passes=False)

_NODE = jax.ShapeDtypeStruct((ACC_N, HID), jnp.float32)
_PART = jax.ShapeDtypeStruct((NC, ACC_N, HID), jnp.float32)


def _rsqrt_nr(d):
  i = plsc.bitcast(d, jnp.int32)
  i = jnp.int32(0x5F3759DF) - lax.shift_right_logical(i, 1)
  y = plsc.bitcast(i, jnp.float32)
  for _ in range(3):
    y = y * (1.5 - 0.5 * d * y * y)
  return y


def _zero_fill(buf, nrows):
  def zrow(i, _):
    buf[i, :] = jnp.zeros((HID,), jnp.float32)
    return 0
  lax.fori_loop(0, nrows, zrow, 0)


def _edge_pass(tab_sh, acc_sh, src_v, dst_v, rows, semg, sems):
  def gis(j, b):
    pltpu.async_copy(tab_sh.at[src_v.at[j]], rows[b], semg[b])

  def gwt(j, b):
    pltpu.make_async_copy(tab_sh.at[src_v.at[j]], rows[b], semg[b]).wait()

  def sis(j, b):
    pltpu.async_copy(rows[b], acc_sh.at[dst_v.at[j]], sems[b], add=True)

  def swt(j, b):
    pltpu.make_async_copy(rows[b], acc_sh.at[dst_v.at[j]], sems[b]).wait()

  gis(0, 0)
  gis(1, 1)
  gis(2, 2)
  gwt(0, 0)
  sis(0, 0)
  gis(3, 3)
  gwt(1, 1)
  sis(1, 1)

  def grp(k, _):
    j0 = 4 * k + 2
    for b2 in range(4):
      j = j0 + b2
      b = (2 + b2) % 4
      bn = (b + 2) % 4
      swt(j - 2, bn)
      gis(j + 2, bn)
      gwt(j, b)
      sis(j, b)
    return 0
  lax.fori_loop(0, (NBLK - 4) // 4, grp, 0)

  for j in (NBLK - 2, NBLK - 1):
    b = j % 4
    gwt(j, b)
    sis(j, b)
  for j in range(NBLK - 4, NBLK):
    swt(j, j % 4)


def _writeout(acc_sh, out_hbm, cid, sid):
  pltpu.sync_copy(acc_sh.at[pl.ds(sid * ROWS_PT, ROWS_PT)],
                  out_hbm.at[cid, pl.ds(sid * ROWS_PT, ROWS_PT)])


def _sc_deg_body(dstp_hbm, out_hbm, acc_sh, dst_v, rows0, stage_v, sem0):
  cid = lax.axis_index("c")
  sid = lax.axis_index("s")
  wid = cid * NS + sid

  pltpu.sync_copy(dstp_hbm.at[wid], dst_v)
  _zero_fill(stage_v, ROWS_PT)
  pltpu.sync_copy(stage_v, acc_sh.at[pl.ds(sid * ROWS_PT, ROWS_PT)])

  def orow(i, _):
    rows0[i, :] = jnp.ones((HID,), jnp.float32)
    return 0
  lax.fori_loop(0, BLK, orow, 0)

  plsc.subcore_barrier()

  def step(j, _):
    pltpu.async_copy(rows0, acc_sh.at[dst_v.at[j]], sem0, add=True)
    return 0
  lax.fori_loop(0, NBLK, step, 0)

  def drain(j, _):
    pltpu.make_async_copy(rows0, acc_sh.at[dst_v.at[j]], sem0).wait()
    return 0
  lax.fori_loop(0, NBLK, drain, 0)

  plsc.subcore_barrier()
  _writeout(acc_sh, out_hbm, cid, sid)


_sc_deg = pl.kernel(
    _sc_deg_body,
    out_type=_PART,
    mesh=_MESH,
    compiler_params=_PARAMS,
    scratch_types=[
        pltpu.VMEM_SHARED((ACC_N, HID), jnp.float32),
        pltpu.VMEM((NBLK, BLK), jnp.int32),
        pltpu.VMEM((BLK, HID), jnp.float32),
        pltpu.VMEM((ROWS_PT, HID), jnp.float32),
        pltpu.SemaphoreType.DMA,
    ],
)


def _sc_l1_body(degp_hbm, z1_hbm, srcp_hbm, dstp_hbm, seg_out, dism_out,
                acc_sh, tab_sh, src_v, dst_v, r0, r1, r2, r3, va, vb, vc,
                g0, g1, g2, g3, s0, s1, s2, s3):
  cid = lax.axis_index("c")
  sid = lax.axis_index("s")
  wid = cid * NS + sid
  sl = pl.ds(sid * ROWS_PT, ROWS_PT)

  pltpu.sync_copy(srcp_hbm.at[wid], src_v)
  pltpu.sync_copy(dstp_hbm.at[wid], dst_v)
  pltpu.sync_copy(degp_hbm.at[0, sl], va)
  pltpu.sync_copy(degp_hbm.at[1, sl], vb)
  pltpu.sync_copy(z1_hbm.at[sl], vc)

  def prep(r, _):
    deg = va[r, :] + vb[r, :] + 1.0
    y = _rsqrt_nr(deg)
    va[r, :] = vc[r, :] * y
    vb[r, :] = y
    return 0
  lax.fori_loop(0, ROWS_PT, prep, 0)

  pltpu.sync_copy(va, tab_sh.at[sl])

  @pl.when(cid == 0)
  def _():
    pltpu.sync_copy(va, acc_sh.at[sl])
    pltpu.sync_copy(vb, dism_out.at[sl])

  @pl.when(cid != 0)
  def _():
    _zero_fill(vc, ROWS_PT)
    pltpu.sync_copy(vc, acc_sh.at[sl])

  plsc.subcore_barrier()
  _edge_pass(tab_sh, acc_sh, src_v, dst_v,
             [r0, r1, r2, r3], [g0, g1, g2, g3], [s0, s1, s2, s3])
  plsc.subcore_barrier()
  _writeout(acc_sh, seg_out, cid, sid)


_EDGE_SCRATCH = (
    [pltpu.VMEM((NBLK, BLK), jnp.int32)] * 2
    + [pltpu.VMEM((BLK, HID), jnp.float32)] * 4
)

_sc_l1 = pl.kernel(
    _sc_l1_body,
    out_type=(_PART, _NODE),
    mesh=_MESH,
    compiler_params=_PARAMS,
    scratch_types=[
        pltpu.VMEM_SHARED((ACC_N, HID), jnp.float32),
        pltpu.VMEM_SHARED((ACC_N, HID), jnp.float32),
    ] + _EDGE_SCRATCH + [
        pltpu.VMEM((ROWS_PT, HID), jnp.float32),
        pltpu.VMEM((ROWS_PT, HID), jnp.float32),
        pltpu.VMEM((ROWS_PT, HID), jnp.float32),
    ] + [pltpu.SemaphoreType.DMA] * 8,
)


def _sc_l2_body(segp_hbm, dism_hbm, b1_hbm, srcp_hbm, dstp_hbm, seg_out,
                acc_sh, tab_sh, src_v, dst_v, r0, r1, r2, r3, va, vb, vc,
                b1_v, g0, g1, g2, g3, s0, s1, s2, s3):
  cid = lax.axis_index("c")
  sid = lax.axis_index("s")
  wid = cid * NS + sid
  sl = pl.ds(sid * ROWS_PT, ROWS_PT)

  pltpu.sync_copy(srcp_hbm.at[wid], src_v)
  pltpu.sync_copy(dstp_hbm.at[wid], dst_v)
  pltpu.sync_copy(segp_hbm.at[0, sl], va)
  pltpu.sync_copy(segp_hbm.at[1, sl], vb)
  pltpu.sync_copy(dism_hbm.at[sl], vc)
  pltpu.sync_copy(b1_hbm, b1_v)

  def prep(r, _):
    y = vc[r, :]
    h = jnp.maximum(y * (va[r, :] + vb[r, :]) + b1_v[...], 0.0)
    va[r, :] = h * y
    return 0
  lax.fori_loop(0, ROWS_PT, prep, 0)

  pltpu.sync_copy(va, tab_sh.at[sl])

  @pl.when(cid == 0)
  def _():
    pltpu.sync_copy(va, acc_sh.at[sl])

  @pl.when(cid != 0)
  def _():
    _zero_fill(vc, ROWS_PT)
    pltpu.sync_copy(vc, acc_sh.at[sl])

  plsc.subcore_barrier()
  _edge_pass(tab_sh, acc_sh, src_v, dst_v,
             [r0, r1, r2, r3], [g0, g1, g2, g3], [s0, s1, s2, s3])
  plsc.subcore_barrier()
  _writeout(acc_sh, seg_out, cid, sid)


_sc_l2 = pl.kernel(
    _sc_l2_body,
    out_type=_PART,
    mesh=_MESH,
    compiler_params=_PARAMS,
    scratch_types=[
        pltpu.VMEM_SHARED((ACC_N, HID), jnp.float32),
        pltpu.VMEM_SHARED((ACC_N, HID), jnp.float32),
    ] + _EDGE_SCRATCH + [
        pltpu.VMEM((ROWS_PT, HID), jnp.float32),
        pltpu.VMEM((ROWS_PT, HID), jnp.float32),
        pltpu.VMEM((ROWS_PT, HID), jnp.float32),
        pltpu.VMEM((HID,), jnp.float32),
    ] + [pltpu.SemaphoreType.DMA] * 8,
)


def _tc_in_body(x_ref, w1_ref, z1_ref):
  z1 = jnp.dot(x_ref[...], w1_ref[...], preferred_element_type=jnp.float32)
  z1_ref[...] = jnp.concatenate(
      [z1, jnp.zeros((ACC_N - N, HID), jnp.float32)], axis=0)


_tc_in = pl.pallas_call(_tc_in_body, out_shape=_NODE)


def _tc_out_body(segp_ref, dism_ref, w2_ref, b2_ref, out_ref):
  agg = dism_ref[...] * (segp_ref[0] + segp_ref[1])
  out_ref[...] = jnp.dot(agg[:N], w2_ref[...],
                         preferred_element_type=jnp.float32) + b2_ref[...]


_tc_out = pl.pallas_call(
    _tc_out_body, out_shape=jax.ShapeDtypeStruct((N, D_OUT), jnp.float32))


def kernel(x, edge_index, W1, b1, W2, b2):
  src = edge_index[0].astype(jnp.int32)
  dst = edge_index[1].astype(jnp.int32)
  npad = PAD_E - E
  src_p = jnp.concatenate([src, jnp.zeros((npad,), jnp.int32)])
  dst_p = jnp.concatenate([dst, jnp.full((npad,), N, jnp.int32)])
  srcp = src_p.reshape(NW, NBLK, BLK)
  dstp = dst_p.reshape(NW, NBLK, BLK)

  degp = _sc_deg(dstp)
  z1 = _tc_in(x, W1)
  seg1p, dism = _sc_l1(degp, z1, srcp, dstp)
  seg2p = _sc_l2(seg1p, dism, b1, srcp, dstp)
  out = _tc_out(seg2p, dism, W2, b2.reshape(1, D_OUT))
  return out


# speedup vs baseline: 1.0885x; 1.0885x over previous
import functools

import jax
import jax.numpy as jnp
from jax import lax
from jax.experimental import pallas as pl
from jax.experimental.pallas import tpu as pltpu
from jax.experimental.pallas import tpu_sc as plsc

N = 10000
E = 320000
D_IN = 128
HID = 16
D_OUT = 128

NC = 2
NS = 16
NW = NC * NS
BLK = 128
NBLK = 80
EPT = NBLK * BLK
E_PT = E // NW
ROWS_PT = 632
ACC_N = ROWS_PT * NS

_MESH = plsc.VectorSubcoreMesh(core_axis_name="c", subcore_axis_name="s")
_PARAMS = pltpu.CompilerParams(use_tc_tiling_on_sc=False, needs_layout_passes=False)

_NODE = jax.ShapeDtypeStruct((ACC_N, HID), jnp.float32)
_PART = jax.ShapeDtypeStruct((NC, ACC_N, HID), jnp.float32)


def _rsqrt_nr(d):
  i = plsc.bitcast(d, jnp.int32)
  i = jnp.int32(0x5F3759DF) - lax.shift_right_logical(i, 1)
  y = plsc.bitcast(i, jnp.float32)
  for _ in range(3):
    y = y * (1.5 - 0.5 * d * y * y)
  return y


def _zero_fill(buf, nrows):
  def zrow(i, _):
    buf[i, :] = jnp.zeros((HID,), jnp.float32)
    return 0
  lax.fori_loop(0, nrows, zrow, 0)


def _load_idx(edge_hbm, row, wid, buf, fill):
  v = jnp.full((16,), fill, jnp.int32)
  for k in range(E_PT, EPT, 16):
    buf[pl.ds(k, 16)] = v
  pltpu.sync_copy(edge_hbm.at[row, pl.ds(wid * E_PT, E_PT)],
                  buf.at[pl.ds(0, E_PT)])


def _edge_pass(tab_sh, acc_sh, src_v, dst_v, rows0, rows1, sem0, sem1):
  def src_at(j):
    return src_v.at[pl.ds(j * BLK, BLK)]

  def dst_at(j):
    return dst_v.at[pl.ds(j * BLK, BLK)]

  pltpu.async_copy(tab_sh.at[src_at(0)], rows0, sem0)

  def pair(k, _):
    j0 = 2 * k
    pltpu.make_async_copy(tab_sh.at[src_at(j0)], rows0, sem0).wait()
    pltpu.async_copy(tab_sh.at[src_at(j0 + 1)], rows1, sem1)
    pltpu.sync_copy(rows0, acc_sh.at[dst_at(j0)], add=True)
    pltpu.make_async_copy(tab_sh.at[src_at(j0 + 1)], rows1, sem1).wait()

    @pl.when(j0 + 2 < NBLK)
    def _():
      pltpu.async_copy(tab_sh.at[src_at(j0 + 2)], rows0, sem0)
    pltpu.sync_copy(rows1, acc_sh.at[dst_at(j0 + 1)], add=True)
    return 0
  lax.fori_loop(0, NBLK // 2, pair, 0)


def _writeout(acc_sh, out_hbm, cid, sid):
  pltpu.sync_copy(acc_sh.at[pl.ds(sid * ROWS_PT, ROWS_PT)],
                  out_hbm.at[cid, pl.ds(sid * ROWS_PT, ROWS_PT)])


def _sc_deg_body(edge_hbm, out_hbm, acc_sh, dst_v, rows0, stage_v, sem0):
  cid = lax.axis_index("c")
  sid = lax.axis_index("s")
  wid = cid * NS + sid

  _load_idx(edge_hbm, 1, wid, dst_v, N)
  _zero_fill(stage_v, ROWS_PT)
  pltpu.sync_copy(stage_v, acc_sh.at[pl.ds(sid * ROWS_PT, ROWS_PT)])

  def orow(i, _):
    rows0[i, :] = jnp.ones((HID,), jnp.float32)
    return 0
  lax.fori_loop(0, BLK, orow, 0)

  plsc.subcore_barrier()

  def step(j, _):
    pltpu.async_copy(rows0, acc_sh.at[dst_v.at[pl.ds(j * BLK, BLK)]],
                     sem0, add=True)
    return 0
  lax.fori_loop(0, NBLK, step, 0)

  def drain(j, _):
    pltpu.make_async_copy(rows0, acc_sh.at[dst_v.at[pl.ds(j * BLK, BLK)]],
                          sem0).wait()
    return 0
  lax.fori_loop(0, NBLK, drain, 0)

  plsc.subcore_barrier()
  _writeout(acc_sh, out_hbm, cid, sid)


_sc_deg = pl.kernel(
    _sc_deg_body,
    out_type=_PART,
    mesh=_MESH,
    compiler_params=_PARAMS,
    scratch_types=[
        pltpu.VMEM_SHARED((ACC_N, HID), jnp.float32),
        pltpu.VMEM((EPT,), jnp.int32),
        pltpu.VMEM((BLK, HID), jnp.float32),
        pltpu.VMEM((ROWS_PT, HID), jnp.float32),
        pltpu.SemaphoreType.DMA,
    ],
)


def _sc_l1_body(degp_hbm, z1_hbm, edge_hbm, seg_out, dism_out,
                acc_sh, tab_sh, src_v, dst_v, r0, r1, va, vb, vc,
                sem0, sem1):
  cid = lax.axis_index("c")
  sid = lax.axis_index("s")
  wid = cid * NS + sid
  sl = pl.ds(sid * ROWS_PT, ROWS_PT)

  _load_idx(edge_hbm, 0, wid, src_v, 0)
  _load_idx(edge_hbm, 1, wid, dst_v, N)
  pltpu.sync_copy(degp_hbm.at[0, sl], va)
  pltpu.sync_copy(degp_hbm.at[1, sl], vb)
  pltpu.sync_copy(z1_hbm.at[sl], vc)

  def prep(r, _):
    deg = va[r, :] + vb[r, :] + 1.0
    y = _rsqrt_nr(deg)
    va[r, :] = vc[r, :] * y
    vb[r, :] = y
    return 0
  lax.fori_loop(0, ROWS_PT, prep, 0)

  pltpu.sync_copy(va, tab_sh.at[sl])

  @pl.when(cid == 0)
  def _():
    pltpu.sync_copy(va, acc_sh.at[sl])
    pltpu.sync_copy(vb, dism_out.at[sl])

  @pl.when(cid != 0)
  def _():
    _zero_fill(vc, ROWS_PT)
    pltpu.sync_copy(vc, acc_sh.at[sl])

  plsc.subcore_barrier()
  _edge_pass(tab_sh, acc_sh, src_v, dst_v, r0, r1, sem0, sem1)
  plsc.subcore_barrier()
  _writeout(acc_sh, seg_out, cid, sid)


_EDGE_SCRATCH = (
    [pltpu.VMEM((EPT,), jnp.int32)] * 2
    + [pltpu.VMEM((BLK, HID), jnp.float32)] * 2
)

_sc_l1 = pl.kernel(
    _sc_l1_body,
    out_type=(_PART, _NODE),
    mesh=_MESH,
    compiler_params=_PARAMS,
    scratch_types=[
        pltpu.VMEM_SHARED((ACC_N, HID), jnp.float32),
        pltpu.VMEM_SHARED((ACC_N, HID), jnp.float32),
    ] + _EDGE_SCRATCH + [
        pltpu.VMEM((ROWS_PT, HID), jnp.float32),
        pltpu.VMEM((ROWS_PT, HID), jnp.float32),
        pltpu.VMEM((ROWS_PT, HID), jnp.float32),
    ] + [pltpu.SemaphoreType.DMA] * 2,
)


def _sc_l2_body(segp_hbm, dism_hbm, b1_hbm, edge_hbm, seg_out,
                acc_sh, tab_sh, src_v, dst_v, r0, r1, va, vb, vc,
                b1_v, sem0, sem1):
  cid = lax.axis_index("c")
  sid = lax.axis_index("s")
  wid = cid * NS + sid
  sl = pl.ds(sid * ROWS_PT, ROWS_PT)

  _load_idx(edge_hbm, 0, wid, src_v, 0)
  _load_idx(edge_hbm, 1, wid, dst_v, N)
  pltpu.sync_copy(segp_hbm.at[0, sl], va)
  pltpu.sync_copy(segp_hbm.at[1, sl], vb)
  pltpu.sync_copy(dism_hbm.at[sl], vc)
  pltpu.sync_copy(b1_hbm, b1_v)

  def prep(r, _):
    y = vc[r, :]
    h = jnp.maximum(y * (va[r, :] + vb[r, :]) + b1_v[...], 0.0)
    va[r, :] = h * y
    return 0
  lax.fori_loop(0, ROWS_PT, prep, 0)

  pltpu.sync_copy(va, tab_sh.at[sl])

  @pl.when(cid == 0)
  def _():
    pltpu.sync_copy(va, acc_sh.at[sl])

  @pl.when(cid != 0)
  def _():
    _zero_fill(vc, ROWS_PT)
    pltpu.sync_copy(vc, acc_sh.at[sl])

  plsc.subcore_barrier()
  _edge_pass(tab_sh, acc_sh, src_v, dst_v, r0, r1, sem0, sem1)
  plsc.subcore_barrier()
  _writeout(acc_sh, seg_out, cid, sid)


_sc_l2 = pl.kernel(
    _sc_l2_body,
    out_type=_PART,
    mesh=_MESH,
    compiler_params=_PARAMS,
    scratch_types=[
        pltpu.VMEM_SHARED((ACC_N, HID), jnp.float32),
        pltpu.VMEM_SHARED((ACC_N, HID), jnp.float32),
    ] + _EDGE_SCRATCH + [
        pltpu.VMEM((ROWS_PT, HID), jnp.float32),
        pltpu.VMEM((ROWS_PT, HID), jnp.float32),
        pltpu.VMEM((ROWS_PT, HID), jnp.float32),
        pltpu.VMEM((HID,), jnp.float32),
    ] + [pltpu.SemaphoreType.DMA] * 2,
)


def _tc_in_body(x_ref, w1_ref, z1_ref):
  z1 = jnp.dot(x_ref[...], w1_ref[...], preferred_element_type=jnp.float32)
  z1_ref[...] = jnp.concatenate(
      [z1, jnp.zeros((ACC_N - N, HID), jnp.float32)], axis=0)


_tc_in = pl.pallas_call(_tc_in_body, out_shape=_NODE)


def _tc_out_body(segp_ref, dism_ref, w2_ref, b2_ref, out_ref):
  agg = dism_ref[...] * (segp_ref[0] + segp_ref[1])
  out_ref[...] = jnp.dot(agg[:N], w2_ref[...],
                         preferred_element_type=jnp.float32) + b2_ref[...]


_tc_out = pl.pallas_call(
    _tc_out_body, out_shape=jax.ShapeDtypeStruct((N, D_OUT), jnp.float32))


def kernel(x, edge_index, W1, b1, W2, b2):
  degp = _sc_deg(edge_index)
  z1 = _tc_in(x, W1)
  seg1p, dism = _sc_l1(degp, z1, edge_index)
  seg2p = _sc_l2(seg1p, dism, b1, edge_index)
  out = _tc_out(seg2p, dism, W2, b2.reshape(1, D_OUT))
  return out


# speedup vs baseline: 1.2575x; 1.1553x over previous
import functools

import jax
import jax.numpy as jnp
from jax import lax
from jax.experimental import pallas as pl
from jax.experimental.pallas import tpu as pltpu
from jax.experimental.pallas import tpu_sc as plsc

N = 10000
E = 320000
D_IN = 128
HID = 16
D_OUT = 128

NC = 2
NS = 16
NW = NC * NS
BLK = 128
NBLK = 80
EPT = NBLK * BLK
E_PT = E // NW
ROWS_PT = 632
ACC_N = ROWS_PT * NS

_MESH = plsc.VectorSubcoreMesh(core_axis_name="c", subcore_axis_name="s")
_PARAMS = pltpu.CompilerParams(use_tc_tiling_on_sc=False, needs_layout_passes=False)

_NODE = jax.ShapeDtypeStruct((ACC_N, HID), jnp.float32)
_PART = jax.ShapeDtypeStruct((NC, ACC_N, HID), jnp.float32)


def _rsqrt_nr(d):
  i = plsc.bitcast(d, jnp.int32)
  i = jnp.int32(0x5F3759DF) - lax.shift_right_logical(i, 1)
  y = plsc.bitcast(i, jnp.float32)
  for _ in range(2):
    y = y * (1.5 - 0.5 * d * y * y)
  return y


def _zero_fill(buf, nrows):
  def zrow(i, _):
    buf[i, :] = jnp.zeros((HID,), jnp.float32)
    return 0
  lax.fori_loop(0, nrows, zrow, 0)


def _idx_copy(edge_hbm, row, wid, buf, fill, sem):
  v = jnp.full((16,), fill, jnp.int32)
  for k in range(E_PT, EPT, 16):
    buf[pl.ds(k, 16)] = v
  return pltpu.async_copy(edge_hbm.at[row, pl.ds(wid * E_PT, E_PT)],
                          buf.at[pl.ds(0, E_PT)], sem)


def _edge_pass(tab_sh, acc_sh, src_v, dst_v, rows0, rows1, sem0, sem1):
  def src_at(j):
    return src_v.at[pl.ds(j * BLK, BLK)]

  def dst_at(j):
    return dst_v.at[pl.ds(j * BLK, BLK)]

  pltpu.async_copy(tab_sh.at[src_at(0)], rows0, sem0)

  def pair(k, _):
    j0 = 2 * k
    pltpu.make_async_copy(tab_sh.at[src_at(j0)], rows0, sem0).wait()
    pltpu.async_copy(tab_sh.at[src_at(j0 + 1)], rows1, sem1)
    pltpu.sync_copy(rows0, acc_sh.at[dst_at(j0)], add=True)
    pltpu.make_async_copy(tab_sh.at[src_at(j0 + 1)], rows1, sem1).wait()

    @pl.when(j0 + 2 < NBLK)
    def _():
      pltpu.async_copy(tab_sh.at[src_at(j0 + 2)], rows0, sem0)
    pltpu.sync_copy(rows1, acc_sh.at[dst_at(j0 + 1)], add=True)
    return 0
  lax.fori_loop(0, NBLK // 2, pair, 0)


def _writeout(acc_sh, out_hbm, cid, sid):
  pltpu.sync_copy(acc_sh.at[pl.ds(sid * ROWS_PT, ROWS_PT)],
                  out_hbm.at[cid, pl.ds(sid * ROWS_PT, ROWS_PT)])


def _sc_deg_body(edge_hbm, out_hbm, acc_sh, dst_v, rows0, stage_v, sem0):
  cid = lax.axis_index("c")
  sid = lax.axis_index("s")
  wid = cid * NS + sid

  cpi = _idx_copy(edge_hbm, 1, wid, dst_v, N, sem0)
  _zero_fill(stage_v, ROWS_PT)
  pltpu.sync_copy(stage_v, acc_sh.at[pl.ds(sid * ROWS_PT, ROWS_PT)])

  def orow(i, _):
    rows0[i, :] = jnp.ones((HID,), jnp.float32)
    return 0
  lax.fori_loop(0, BLK, orow, 0)

  cpi.wait()
  plsc.subcore_barrier()

  def step(j, _):
    pltpu.async_copy(rows0, acc_sh.at[dst_v.at[pl.ds(j * BLK, BLK)]],
                     sem0, add=True)
    return 0
  lax.fori_loop(0, NBLK, step, 0)

  def drain(j, _):
    pltpu.make_async_copy(rows0, acc_sh.at[dst_v.at[pl.ds(j * BLK, BLK)]],
                          sem0).wait()
    return 0
  lax.fori_loop(0, NBLK, drain, 0)

  plsc.subcore_barrier()
  _writeout(acc_sh, out_hbm, cid, sid)


_sc_deg = pl.kernel(
    _sc_deg_body,
    out_type=_PART,
    mesh=_MESH,
    compiler_params=_PARAMS,
    scratch_types=[
        pltpu.VMEM_SHARED((ACC_N, HID), jnp.float32),
        pltpu.VMEM((EPT,), jnp.int32),
        pltpu.VMEM((BLK, HID), jnp.float32),
        pltpu.VMEM((ROWS_PT, HID), jnp.float32),
        pltpu.SemaphoreType.DMA,
    ],
)


def _sc_l1_body(degp_hbm, z1_hbm, edge_hbm, seg_out, dism_out,
                acc_sh, tab_sh, src_v, dst_v, r0, r1, va, vb, vc,
                sem0, sem1):
  cid = lax.axis_index("c")
  sid = lax.axis_index("s")
  wid = cid * NS + sid
  sl = pl.ds(sid * ROWS_PT, ROWS_PT)

  ci0 = _idx_copy(edge_hbm, 0, wid, src_v, 0, sem0)
  ci1 = _idx_copy(edge_hbm, 1, wid, dst_v, N, sem0)
  ca = pltpu.async_copy(degp_hbm.at[0, sl], va, sem1)
  cb = pltpu.async_copy(degp_hbm.at[1, sl], vb, sem1)
  cc = pltpu.async_copy(z1_hbm.at[sl], vc, sem1)
  ca.wait()
  cb.wait()
  cc.wait()

  def prep(q, _):
    for u in range(8):
      r = q * 8 + u
      deg = va[r, :] + vb[r, :] + 1.0
      y = _rsqrt_nr(deg)
      va[r, :] = vc[r, :] * y
      vb[r, :] = y
    return 0
  lax.fori_loop(0, ROWS_PT // 8, prep, 0)

  pltpu.sync_copy(va, tab_sh.at[sl])

  @pl.when(cid == 0)
  def _():
    pltpu.sync_copy(va, acc_sh.at[sl])
    pltpu.sync_copy(vb, dism_out.at[sl])

  @pl.when(cid != 0)
  def _():
    _zero_fill(vc, ROWS_PT)
    pltpu.sync_copy(vc, acc_sh.at[sl])

  ci0.wait()
  ci1.wait()
  plsc.subcore_barrier()
  _edge_pass(tab_sh, acc_sh, src_v, dst_v, r0, r1, sem0, sem1)
  plsc.subcore_barrier()
  _writeout(acc_sh, seg_out, cid, sid)


_EDGE_SCRATCH = (
    [pltpu.VMEM((EPT,), jnp.int32)] * 2
    + [pltpu.VMEM((BLK, HID), jnp.float32)] * 2
)

_sc_l1 = pl.kernel(
    _sc_l1_body,
    out_type=(_PART, _NODE),
    mesh=_MESH,
    compiler_params=_PARAMS,
    scratch_types=[
        pltpu.VMEM_SHARED((ACC_N, HID), jnp.float32),
        pltpu.VMEM_SHARED((ACC_N, HID), jnp.float32),
    ] + _EDGE_SCRATCH + [
        pltpu.VMEM((ROWS_PT, HID), jnp.float32),
        pltpu.VMEM((ROWS_PT, HID), jnp.float32),
        pltpu.VMEM((ROWS_PT, HID), jnp.float32),
    ] + [pltpu.SemaphoreType.DMA] * 2,
)


def _sc_l2_body(segp_hbm, dism_hbm, b1_hbm, edge_hbm, seg_out,
                acc_sh, tab_sh, src_v, dst_v, r0, r1, va, vb, vc,
                b1_v, sem0, sem1):
  cid = lax.axis_index("c")
  sid = lax.axis_index("s")
  wid = cid * NS + sid
  sl = pl.ds(sid * ROWS_PT, ROWS_PT)

  ci0 = _idx_copy(edge_hbm, 0, wid, src_v, 0, sem0)
  ci1 = _idx_copy(edge_hbm, 1, wid, dst_v, N, sem0)
  ca = pltpu.async_copy(segp_hbm.at[0, sl], va, sem1)
  cb = pltpu.async_copy(segp_hbm.at[1, sl], vb, sem1)
  cc = pltpu.async_copy(dism_hbm.at[sl], vc, sem1)
  pltpu.sync_copy(b1_hbm, b1_v)
  ca.wait()
  cb.wait()
  cc.wait()

  def prep(q, _):
    for u in range(8):
      r = q * 8 + u
      y = vc[r, :]
      h = jnp.maximum(y * (va[r, :] + vb[r, :]) + b1_v[...], 0.0)
      va[r, :] = h * y
    return 0
  lax.fori_loop(0, ROWS_PT // 8, prep, 0)

  pltpu.sync_copy(va, tab_sh.at[sl])

  @pl.when(cid == 0)
  def _():
    pltpu.sync_copy(va, acc_sh.at[sl])

  @pl.when(cid != 0)
  def _():
    _zero_fill(vc, ROWS_PT)
    pltpu.sync_copy(vc, acc_sh.at[sl])

  ci0.wait()
  ci1.wait()
  plsc.subcore_barrier()
  _edge_pass(tab_sh, acc_sh, src_v, dst_v, r0, r1, sem0, sem1)
  plsc.subcore_barrier()
  _writeout(acc_sh, seg_out, cid, sid)


_sc_l2 = pl.kernel(
    _sc_l2_body,
    out_type=_PART,
    mesh=_MESH,
    compiler_params=_PARAMS,
    scratch_types=[
        pltpu.VMEM_SHARED((ACC_N, HID), jnp.float32),
        pltpu.VMEM_SHARED((ACC_N, HID), jnp.float32),
    ] + _EDGE_SCRATCH + [
        pltpu.VMEM((ROWS_PT, HID), jnp.float32),
        pltpu.VMEM((ROWS_PT, HID), jnp.float32),
        pltpu.VMEM((ROWS_PT, HID), jnp.float32),
        pltpu.VMEM((HID,), jnp.float32),
    ] + [pltpu.SemaphoreType.DMA] * 2,
)


def _tc_in_body(x_ref, w1_ref, z1_ref):
  z1 = jnp.dot(x_ref[...], w1_ref[...], preferred_element_type=jnp.float32)
  z1_ref[...] = jnp.concatenate(
      [z1, jnp.zeros((ACC_N - N, HID), jnp.float32)], axis=0)


_tc_in = pl.pallas_call(_tc_in_body, out_shape=_NODE)


def _tc_out_body(segp_ref, dism_ref, w2_ref, b2_ref, out_ref):
  agg = dism_ref[...] * (segp_ref[0] + segp_ref[1])
  out_ref[...] = jnp.dot(agg[:N], w2_ref[...],
                         preferred_element_type=jnp.float32) + b2_ref[...]


_tc_out = pl.pallas_call(
    _tc_out_body, out_shape=jax.ShapeDtypeStruct((N, D_OUT), jnp.float32))


def kernel(x, edge_index, W1, b1, W2, b2):
  degp = _sc_deg(edge_index)
  z1 = _tc_in(x, W1)
  seg1p, dism = _sc_l1(degp, z1, edge_index)
  seg2p = _sc_l2(seg1p, dism, b1, edge_index)
  out = _tc_out(seg2p, dism, W2, b2.reshape(1, D_OUT))
  return out


# speedup vs baseline: 1.2903x; 1.0261x over previous
import functools

import jax
import jax.numpy as jnp
from jax import lax
from jax.experimental import pallas as pl
from jax.experimental.pallas import tpu as pltpu
from jax.experimental.pallas import tpu_sc as plsc

N = 10000
E = 320000
D_IN = 128
HID = 16
D_OUT = 128

NC = 2
NS = 16
NW = NC * NS
BLK = 128
NBLK = 80
SBLK = 512
NSB = 20
EPT = NBLK * BLK
E_PT = E // NW
ROWS_PT = 632
ACC_N = ROWS_PT * NS

_MESH = plsc.VectorSubcoreMesh(core_axis_name="c", subcore_axis_name="s")
_PARAMS = pltpu.CompilerParams(use_tc_tiling_on_sc=False, needs_layout_passes=False)

_NODE = jax.ShapeDtypeStruct((ACC_N, HID), jnp.float32)
_PART = jax.ShapeDtypeStruct((NC, ACC_N, HID), jnp.float32)


def _rsqrt_nr(d):
  i = plsc.bitcast(d, jnp.int32)
  i = jnp.int32(0x5F3759DF) - lax.shift_right_logical(i, 1)
  y = plsc.bitcast(i, jnp.float32)
  for _ in range(2):
    y = y * (1.5 - 0.5 * d * y * y)
  return y


def _zero_fill(buf, nrows):
  def zrow(i, _):
    buf[i, :] = jnp.zeros((HID,), jnp.float32)
    return 0
  lax.fori_loop(0, nrows, zrow, 0)


def _idx_copy(edge_hbm, row, wid, buf, fill, sem):
  v = jnp.full((16,), fill, jnp.int32)
  for k in range(E_PT, EPT, 16):
    buf[pl.ds(k, 16)] = v
  return pltpu.async_copy(edge_hbm.at[row, pl.ds(wid * E_PT, E_PT)],
                          buf.at[pl.ds(0, E_PT)], sem)


def _edge_pass(tab_sh, acc_sh, src_v, dst_v, rows0, rows1, sem0, sem1):
  def gath(sb, q, buf, sem):
    j = sb * 4 + q
    return pltpu.async_copy(tab_sh.at[src_v.at[pl.ds(j * BLK, BLK)]],
                            buf.at[pl.ds(q * BLK, BLK)], sem)

  def gwait(sb, q, buf, sem):
    j = sb * 4 + q
    pltpu.make_async_copy(tab_sh.at[src_v.at[pl.ds(j * BLK, BLK)]],
                          buf.at[pl.ds(q * BLK, BLK)], sem).wait()

  def scat(sb, buf):
    pltpu.sync_copy(buf, acc_sh.at[dst_v.at[pl.ds(sb * SBLK, SBLK)]],
                    add=True)

  for q in range(4):
    gath(0, q, rows0, sem0)

  def pair(k, _):
    sb0 = 2 * k
    for q in range(4):
      gwait(sb0, q, rows0, sem0)
    for q in range(4):
      gath(sb0 + 1, q, rows1, sem1)
    scat(sb0, rows0)
    for q in range(4):
      gwait(sb0 + 1, q, rows1, sem1)

    @pl.when(sb0 + 2 < NSB)
    def _():
      for q in range(4):
        gath(sb0 + 2, q, rows0, sem0)
    scat(sb0 + 1, rows1)
    return 0
  lax.fori_loop(0, NSB // 2, pair, 0)


def _writeout(acc_sh, out_hbm, cid, sid):
  pltpu.sync_copy(acc_sh.at[pl.ds(sid * ROWS_PT, ROWS_PT)],
                  out_hbm.at[cid, pl.ds(sid * ROWS_PT, ROWS_PT)])


def _sc_deg_body(edge_hbm, out_hbm, acc_sh, dst_v, rows0, stage_v, sem0):
  cid = lax.axis_index("c")
  sid = lax.axis_index("s")
  wid = cid * NS + sid

  cpi = _idx_copy(edge_hbm, 1, wid, dst_v, N, sem0)
  _zero_fill(stage_v, ROWS_PT)
  pltpu.sync_copy(stage_v, acc_sh.at[pl.ds(sid * ROWS_PT, ROWS_PT)])

  def orow(i, _):
    rows0[i, :] = jnp.ones((HID,), jnp.float32)
    return 0
  lax.fori_loop(0, BLK, orow, 0)

  cpi.wait()
  plsc.subcore_barrier()

  def step(j, _):
    pltpu.async_copy(rows0, acc_sh.at[dst_v.at[pl.ds(j * BLK, BLK)]],
                     sem0, add=True)
    return 0
  lax.fori_loop(0, NBLK, step, 0)

  def drain(j, _):
    pltpu.make_async_copy(rows0, acc_sh.at[dst_v.at[pl.ds(j * BLK, BLK)]],
                          sem0).wait()
    return 0
  lax.fori_loop(0, NBLK, drain, 0)

  plsc.subcore_barrier()
  _writeout(acc_sh, out_hbm, cid, sid)


_sc_deg = pl.kernel(
    _sc_deg_body,
    out_type=_PART,
    mesh=_MESH,
    compiler_params=_PARAMS,
    scratch_types=[
        pltpu.VMEM_SHARED((ACC_N, HID), jnp.float32),
        pltpu.VMEM((EPT,), jnp.int32),
        pltpu.VMEM((BLK, HID), jnp.float32),
        pltpu.VMEM((ROWS_PT, HID), jnp.float32),
        pltpu.SemaphoreType.DMA,
    ],
)


def _sc_l1_body(degp_hbm, z1_hbm, edge_hbm, seg_out, dism_out,
                acc_sh, tab_sh, src_v, dst_v, r0, r1, va, vb, vc,
                sem0, sem1):
  cid = lax.axis_index("c")
  sid = lax.axis_index("s")
  wid = cid * NS + sid
  sl = pl.ds(sid * ROWS_PT, ROWS_PT)

  ci0 = _idx_copy(edge_hbm, 0, wid, src_v, 0, sem0)
  ci1 = _idx_copy(edge_hbm, 1, wid, dst_v, N, sem0)
  ca = pltpu.async_copy(degp_hbm.at[0, sl], va, sem1)
  cb = pltpu.async_copy(degp_hbm.at[1, sl], vb, sem1)
  cc = pltpu.async_copy(z1_hbm.at[sl], vc, sem1)
  ca.wait()
  cb.wait()
  cc.wait()

  def prep(q, _):
    for u in range(8):
      r = q * 8 + u
      deg = va[r, :] + vb[r, :] + 1.0
      y = _rsqrt_nr(deg)
      va[r, :] = vc[r, :] * y
      vb[r, :] = y
    return 0
  lax.fori_loop(0, ROWS_PT // 8, prep, 0)

  pltpu.sync_copy(va, tab_sh.at[sl])

  @pl.when(cid == 0)
  def _():
    pltpu.sync_copy(va, acc_sh.at[sl])
    pltpu.sync_copy(vb, dism_out.at[sl])

  @pl.when(cid != 0)
  def _():
    _zero_fill(vc, ROWS_PT)
    pltpu.sync_copy(vc, acc_sh.at[sl])

  ci0.wait()
  ci1.wait()
  plsc.subcore_barrier()
  _edge_pass(tab_sh, acc_sh, src_v, dst_v, r0, r1, sem0, sem1)
  plsc.subcore_barrier()
  _writeout(acc_sh, seg_out, cid, sid)


_EDGE_SCRATCH = (
    [pltpu.VMEM((EPT,), jnp.int32)] * 2
    + [pltpu.VMEM((SBLK, HID), jnp.float32)] * 2
)

_sc_l1 = pl.kernel(
    _sc_l1_body,
    out_type=(_PART, _NODE),
    mesh=_MESH,
    compiler_params=_PARAMS,
    scratch_types=[
        pltpu.VMEM_SHARED((ACC_N, HID), jnp.float32),
        pltpu.VMEM_SHARED((ACC_N, HID), jnp.float32),
    ] + _EDGE_SCRATCH + [
        pltpu.VMEM((ROWS_PT, HID), jnp.float32),
        pltpu.VMEM((ROWS_PT, HID), jnp.float32),
        pltpu.VMEM((ROWS_PT, HID), jnp.float32),
    ] + [pltpu.SemaphoreType.DMA] * 2,
)


def _sc_l2_body(segp_hbm, dism_hbm, b1_hbm, edge_hbm, seg_out,
                acc_sh, tab_sh, src_v, dst_v, r0, r1, va, vb, vc,
                b1_v, sem0, sem1):
  cid = lax.axis_index("c")
  sid = lax.axis_index("s")
  wid = cid * NS + sid
  sl = pl.ds(sid * ROWS_PT, ROWS_PT)

  ci0 = _idx_copy(edge_hbm, 0, wid, src_v, 0, sem0)
  ci1 = _idx_copy(edge_hbm, 1, wid, dst_v, N, sem0)
  ca = pltpu.async_copy(segp_hbm.at[0, sl], va, sem1)
  cb = pltpu.async_copy(segp_hbm.at[1, sl], vb, sem1)
  cc = pltpu.async_copy(dism_hbm.at[sl], vc, sem1)
  pltpu.sync_copy(b1_hbm, b1_v)
  ca.wait()
  cb.wait()
  cc.wait()

  def prep(q, _):
    for u in range(8):
      r = q * 8 + u
      y = vc[r, :]
      h = jnp.maximum(y * (va[r, :] + vb[r, :]) + b1_v[...], 0.0)
      va[r, :] = h * y
    return 0
  lax.fori_loop(0, ROWS_PT // 8, prep, 0)

  pltpu.sync_copy(va, tab_sh.at[sl])

  @pl.when(cid == 0)
  def _():
    pltpu.sync_copy(va, acc_sh.at[sl])

  @pl.when(cid != 0)
  def _():
    _zero_fill(vc, ROWS_PT)
    pltpu.sync_copy(vc, acc_sh.at[sl])

  ci0.wait()
  ci1.wait()
  plsc.subcore_barrier()
  _edge_pass(tab_sh, acc_sh, src_v, dst_v, r0, r1, sem0, sem1)
  plsc.subcore_barrier()
  _writeout(acc_sh, seg_out, cid, sid)


_sc_l2 = pl.kernel(
    _sc_l2_body,
    out_type=_PART,
    mesh=_MESH,
    compiler_params=_PARAMS,
    scratch_types=[
        pltpu.VMEM_SHARED((ACC_N, HID), jnp.float32),
        pltpu.VMEM_SHARED((ACC_N, HID), jnp.float32),
    ] + _EDGE_SCRATCH + [
        pltpu.VMEM((ROWS_PT, HID), jnp.float32),
        pltpu.VMEM((ROWS_PT, HID), jnp.float32),
        pltpu.VMEM((ROWS_PT, HID), jnp.float32),
        pltpu.VMEM((HID,), jnp.float32),
    ] + [pltpu.SemaphoreType.DMA] * 2,
)


def _tc_in_body(x_ref, w1_ref, z1_ref):
  z1 = jnp.dot(x_ref[...], w1_ref[...], preferred_element_type=jnp.float32)
  z1_ref[...] = jnp.concatenate(
      [z1, jnp.zeros((ACC_N - N, HID), jnp.float32)], axis=0)


_tc_in = pl.pallas_call(_tc_in_body, out_shape=_NODE)


def _tc_out_body(segp_ref, dism_ref, w2_ref, b2_ref, out_ref):
  agg = dism_ref[...] * (segp_ref[0] + segp_ref[1])
  out_ref[...] = jnp.dot(agg[:N], w2_ref[...],
                         preferred_element_type=jnp.float32) + b2_ref[...]


_tc_out = pl.pallas_call(
    _tc_out_body, out_shape=jax.ShapeDtypeStruct((N, D_OUT), jnp.float32))


def kernel(x, edge_index, W1, b1, W2, b2):
  degp = _sc_deg(edge_index)
  z1 = _tc_in(x, W1)
  seg1p, dism = _sc_l1(degp, z1, edge_index)
  seg2p = _sc_l2(seg1p, dism, b1, edge_index)
  out = _tc_out(seg2p, dism, W2, b2.reshape(1, D_OUT))
  return out
